# baseline (speedup 1.0000x reference)
import jax
import jax.numpy as jnp
from jax import lax
from jax.experimental import pallas as pl
from jax.experimental.pallas import tpu as pltpu
from jax.experimental.pallas import tpu_sc as plsc

B = 16384
L = 20
D = 64
YD = 4
OD = D + YD
BL = B * L

NC = 2
NS = 16
NW = NC * NS
ROWS_W = BL // NW
CHUNK = 256
NCHUNK = ROWS_W // CHUNK
GSUB = 128
NG = CHUNK // GSUB
LANES = 16


def _body(table, idxf, yawf, cntrow, lrow,
          feat_out, mask_out,
          idx_v, emb_v, yaw_v, cnt_v, l_v, mrow_v, mout_v, out_v,
          gsem0, gsem1, isem0, isem1, osem0, osem1):
  wid = lax.axis_index("s") * NC + lax.axis_index("c")
  w0 = wid * ROWS_W

  iota = lax.iota(jnp.int32, LANES)
  yaw_row_pat = lax.shift_right_logical(iota, 2)
  yaw_dst_pat = 68 * yaw_row_pat + 64 + lax.bitwise_and(iota, 3)

  sems = ((gsem0, isem0, osem0), (gsem1, isem1, osem1))

  def gather_copies(c, s):
    gsem = sems[s][0]
    return [
        pltpu.make_async_copy(
            table.at[idx_v.at[s, pl.ds(j * GSUB, GSUB)]],
            emb_v.at[s, pl.ds(j * GSUB, GSUB)], gsem)
        for j in range(NG)
    ]

  def input_copies(c, s):
    base = w0 + c * CHUNK
    isem = sems[s][1]
    return [
        pltpu.make_async_copy(yawf.at[pl.ds(base * YD, CHUNK * YD)],
                              yaw_v.at[s], isem),
        pltpu.make_async_copy(cntrow.at[pl.ds(base, CHUNK)],
                              cnt_v.at[s], isem),
        pltpu.make_async_copy(lrow.at[pl.ds(base, CHUNK)],
                              l_v.at[s], isem),
    ]

  def output_copies(c, s):
    base = w0 + c * CHUNK
    osem = sems[s][2]
    return [
        pltpu.make_async_copy(out_v.at[s],
                              feat_out.at[pl.ds(base * OD, CHUNK * OD)], osem),
        pltpu.make_async_copy(mout_v.at[s],
                              mask_out.at[pl.ds(base, CHUNK)], osem),
    ]

  def issue(c, s):
    base = w0 + c * CHUNK
    pltpu.sync_copy(idxf.at[pl.ds(base, CHUNK)], idx_v.at[s])
    for cp in gather_copies(c, s):
      cp.start()
    for cp in input_copies(c, s):
      cp.start()

  def drain_out(c, s):
    for cp in output_copies(c, s):
      cp.wait()

  def process(c, s):
    for cp in input_copies(c, s):
      cp.wait()

    def mask_body(j, _):
      lv = l_v[s, pl.ds(j * LANES, LANES)]
      cv = cnt_v[s, pl.ds(j * LANES, LANES)]
      valid = lv < cv
      mrow_v[s, pl.ds(j * LANES, LANES)] = jnp.where(valid, 1.0, 0.0).astype(jnp.float32)
      mout_v[s, pl.ds(j * LANES, LANES)] = jnp.where(valid, 0, 1).astype(jnp.int32)
      return 0
    lax.fori_loop(0, CHUNK // LANES, mask_body, 0, unroll=4)

    for cp in gather_copies(c, s):
      cp.wait()

    def row_body(r, _):
      m = plsc.load_gather(mrow_v.at[s], [jnp.full((LANES,), r, jnp.int32)])
      for k in range(D // LANES):
        v = emb_v[s, r, pl.ds(k * LANES, LANES)]
        out_v[s, pl.ds(r * OD + k * LANES, LANES)] = v * m
      return 0
    lax.fori_loop(0, CHUNK, row_body, 0, unroll=2)

    def yaw_body(j, _):
      rows = 4 * j + yaw_row_pat
      m = plsc.load_gather(mrow_v.at[s], [rows])
      v = yaw_v[s, pl.ds(j * LANES, LANES)]
      plsc.store_scatter(out_v.at[s], [272 * j + yaw_dst_pat], v * m)
      return 0
    lax.fori_loop(0, CHUNK * YD // LANES, yaw_body, 0, unroll=4)

    for cp in output_copies(c, s):
      cp.start()

  issue(0, 0)

  def loop_body(c2, _):
    a = 2 * c2
    issue(a + 1, 1)

    @pl.when(c2 > 0)
    def _():
      drain_out(a - 2, 0)
    process(a, 0)

    @pl.when(c2 < NCHUNK // 2 - 1)
    def _():
      issue(a + 2, 0)

    @pl.when(c2 > 0)
    def _():
      drain_out(a - 1, 1)
    process(a + 1, 1)
    return 0

  lax.fori_loop(0, NCHUNK // 2, loop_body, 0)
  drain_out(NCHUNK - 2, 0)
  drain_out(NCHUNK - 1, 1)


@jax.jit
def _run(table, idxf, yawf, cntrow, lrow):
  mesh = plsc.VectorSubcoreMesh(core_axis_name="c", subcore_axis_name="s",
                                num_cores=NC, num_subcores=NS)
  f = pl.kernel(
      _body,
      out_type=(
          jax.ShapeDtypeStruct((BL * OD,), jnp.float32),
          jax.ShapeDtypeStruct((BL,), jnp.int32),
      ),
      mesh=mesh,
      compiler_params=pltpu.CompilerParams(use_tc_tiling_on_sc=False,
                                           needs_layout_passes=False),
      scratch_types=[
          pltpu.VMEM((2, CHUNK), jnp.int32),
          pltpu.VMEM((2, CHUNK, D), jnp.float32),
          pltpu.VMEM((2, CHUNK * YD), jnp.float32),
          pltpu.VMEM((2, CHUNK), jnp.int32),
          pltpu.VMEM((2, CHUNK), jnp.int32),
          pltpu.VMEM((2, CHUNK), jnp.float32),
          pltpu.VMEM((2, CHUNK), jnp.int32),
          pltpu.VMEM((2, CHUNK * OD), jnp.float32),
          pltpu.SemaphoreType.DMA,
          pltpu.SemaphoreType.DMA,
          pltpu.SemaphoreType.DMA,
          pltpu.SemaphoreType.DMA,
          pltpu.SemaphoreType.DMA,
          pltpu.SemaphoreType.DMA,
      ],
  )
  return f(table, idxf, yawf, cntrow, lrow)


def kernel(indices, yaw_bits, valid_counts, table):
  idxf = indices.reshape(-1)
  yawf = yaw_bits.reshape(-1)
  cntrow = jnp.repeat(valid_counts.astype(jnp.int32), L)
  lrow = jnp.tile(jnp.arange(L, dtype=jnp.int32), B)
  feat_flat, mask_i = _run(table, idxf, yawf, cntrow, lrow)
  features = feat_flat.reshape(B, L, OD)
  mask = mask_i.reshape(B, L).astype(bool)
  return features, mask


# speedup vs baseline: 1.0076x; 1.0076x over previous
import jax
import jax.numpy as jnp
from jax import lax
from jax.experimental import pallas as pl
from jax.experimental.pallas import tpu as pltpu
from jax.experimental.pallas import tpu_sc as plsc

B = 16384
L = 20
D = 64
YD = 4
OD = D + YD
BL = B * L

NC = 2
NS = 16
NW = NC * NS
ROWS_W = BL // NW
CHUNK = 256
NCHUNK = ROWS_W // CHUNK
GSUB = 128
NG = CHUNK // GSUB
LANES = 16
NGRP = CHUNK // LANES


_GDN = lax.GatherDimensionNumbers(
    offset_dims=(), collapsed_slice_dims=(0,), start_index_map=(0,))


def _lane_gather(v, idx):
  return lax.gather(v, idx[:, None], _GDN, (1,),
                    mode=lax.GatherScatterMode.PROMISE_IN_BOUNDS)


def _bcast(v, i):
  return _lane_gather(v, jnp.full((LANES,), i, jnp.int32))


def _body(table, idxf, yawf, cntrow, lrow,
          feat_out, mask_out,
          idx_v, emb_v, yaw_v, cnt_v, l_v, mout_v, out_v,
          gsem0, gsem1, isem0, isem1, osem0, osem1, xsem0, xsem1):
  wid = lax.axis_index("s") * NC + lax.axis_index("c")
  w0 = wid * ROWS_W

  iota = lax.iota(jnp.int32, LANES)
  yaw_row_pat = lax.shift_right_logical(iota, 2)
  yaw_dst_pat = 68 * yaw_row_pat + 64 + lax.bitwise_and(iota, 3)

  gsems = (gsem0, gsem1)
  isems = (isem0, isem1)
  osems = (osem0, osem1)
  xsems = (xsem0, xsem1)

  def idx_copy(c, s):
    base = w0 + c * CHUNK
    return pltpu.make_async_copy(idxf.at[pl.ds(base, CHUNK)],
                                 idx_v.at[s], xsems[s])

  def gather_copies(c, s):
    return [
        pltpu.make_async_copy(
            table.at[idx_v.at[s, pl.ds(j * GSUB, GSUB)]],
            emb_v.at[s, pl.ds(j * GSUB, GSUB)], gsems[s])
        for j in range(NG)
    ]

  def input_copies(c, s):
    base = w0 + c * CHUNK
    return [
        pltpu.make_async_copy(yawf.at[pl.ds(base * YD, CHUNK * YD)],
                              yaw_v.at[s], isems[s]),
        pltpu.make_async_copy(cntrow.at[pl.ds(base, CHUNK)],
                              cnt_v.at[s], isems[s]),
        pltpu.make_async_copy(lrow.at[pl.ds(base, CHUNK)],
                              l_v.at[s], isems[s]),
    ]

  def output_copies(c, s):
    base = w0 + c * CHUNK
    return [
        pltpu.make_async_copy(out_v.at[s],
                              feat_out.at[pl.ds(base * OD, CHUNK * OD)],
                              osems[s]),
        pltpu.make_async_copy(mout_v.at[s],
                              mask_out.at[pl.ds(base, CHUNK)], osems[s]),
    ]

  def stage_idx(c, s):
    idx_copy(c, s).start()

  def fire(c, s):
    idx_copy(c, s).wait()
    for cp in gather_copies(c, s):
      cp.start()
    for cp in input_copies(c, s):
      cp.start()

  def drain_out(c, s):
    for cp in output_copies(c, s):
      cp.wait()

  def compute(c, s):
    for cp in input_copies(c, s):
      cp.wait()
    for cp in gather_copies(c, s):
      cp.wait()

    def grp_body(g, _):
      lv = l_v[s, pl.ds(g * LANES, LANES)]
      cv = cnt_v[s, pl.ds(g * LANES, LANES)]
      valid = lv < cv
      m16 = jnp.where(valid, 1.0, 0.0).astype(jnp.float32)
      mout_v[s, pl.ds(g * LANES, LANES)] = jnp.where(valid, 0, 1).astype(jnp.int32)
      o0 = g * (LANES * OD)
      e0 = g * LANES
      for i in range(LANES):
        m = _bcast(m16, i)
        for k in range(D // LANES):
          v = emb_v[s, e0 + i, pl.ds(k * LANES, LANES)]
          out_v[s, pl.ds(o0 + i * OD + k * LANES, LANES)] = v * m
      for jj in range(4):
        my = _lane_gather(m16, 4 * jj + yaw_row_pat)
        v = yaw_v[s, pl.ds(g * (4 * LANES) + jj * LANES, LANES)]
        plsc.store_scatter(out_v.at[s], [o0 + 272 * jj + yaw_dst_pat], v * my)
      return 0
    lax.fori_loop(0, NGRP, grp_body, 0)

    for cp in output_copies(c, s):
      cp.start()

  stage_idx(0, 0)
  fire(0, 0)
  stage_idx(1, 1)

  def loop_body(c2, _):
    a = 2 * c2
    fire(a + 1, 1)

    @pl.when(c2 > 0)
    def _():
      drain_out(a - 2, 0)
    compute(a, 0)

    @pl.when(c2 < NCHUNK // 2 - 1)
    def _():
      stage_idx(a + 2, 0)
      fire(a + 2, 0)

    @pl.when(c2 > 0)
    def _():
      drain_out(a - 1, 1)
    compute(a + 1, 1)

    @pl.when(c2 < NCHUNK // 2 - 1)
    def _():
      stage_idx(a + 3, 1)
    return 0

  lax.fori_loop(0, NCHUNK // 2, loop_body, 0)
  drain_out(NCHUNK - 2, 0)
  drain_out(NCHUNK - 1, 1)


@jax.jit
def _run(table, idxf, yawf, cntrow, lrow):
  mesh = plsc.VectorSubcoreMesh(core_axis_name="c", subcore_axis_name="s",
                                num_cores=NC, num_subcores=NS)
  f = pl.kernel(
      _body,
      out_type=(
          jax.ShapeDtypeStruct((BL * OD,), jnp.float32),
          jax.ShapeDtypeStruct((BL,), jnp.int32),
      ),
      mesh=mesh,
      compiler_params=pltpu.CompilerParams(use_tc_tiling_on_sc=False,
                                           needs_layout_passes=False),
      scratch_types=[
          pltpu.VMEM((2, CHUNK), jnp.int32),
          pltpu.VMEM((2, CHUNK, D), jnp.float32),
          pltpu.VMEM((2, CHUNK * YD), jnp.float32),
          pltpu.VMEM((2, CHUNK), jnp.int32),
          pltpu.VMEM((2, CHUNK), jnp.int32),
          pltpu.VMEM((2, CHUNK), jnp.int32),
          pltpu.VMEM((2, CHUNK * OD), jnp.float32),
          pltpu.SemaphoreType.DMA,
          pltpu.SemaphoreType.DMA,
          pltpu.SemaphoreType.DMA,
          pltpu.SemaphoreType.DMA,
          pltpu.SemaphoreType.DMA,
          pltpu.SemaphoreType.DMA,
          pltpu.SemaphoreType.DMA,
          pltpu.SemaphoreType.DMA,
      ],
  )
  return f(table, idxf, yawf, cntrow, lrow)


def kernel(indices, yaw_bits, valid_counts, table):
  idxf = indices.reshape(-1)
  yawf = yaw_bits.reshape(-1)
  cntrow = jnp.repeat(valid_counts.astype(jnp.int32), L)
  lrow = jnp.tile(jnp.arange(L, dtype=jnp.int32), B)
  feat_flat, mask_i = _run(table, idxf, yawf, cntrow, lrow)
  features = feat_flat.reshape(B, L, OD)
  mask = mask_i.reshape(B, L).astype(bool)
  return features, mask
